# Initial kernel scaffold; baseline (speedup 1.0000x reference)
import jax
import jax.numpy as jnp
from jax.experimental import pallas as pl


def kernel(_input, neigh_tab, batch_nodes, weight):
    raise NotImplementedError("write your pallas kernel here")


# speedup vs baseline: 2.3218x; 2.3218x over previous
import functools

import jax
import jax.numpy as jnp
from jax import lax
from jax.experimental import pallas as pl
from jax.experimental.pallas import tpu as pltpu
from jax.experimental.pallas import tpu_sc as plsc

N_NODES = 100000
D = 128
K = 10
B = 50000

NC = 2
NS = 16
NW = NC * NS

C = 128
NCHUNK = 13
BPW = C * NCHUNK
BP = BPW * NW

KPAD = 128

ROWS_TC = 512

_MESH = dict(core_axis_name="c", subcore_axis_name="s",
             num_cores=NC, num_subcores=NS)


def _stage1_body(inp_hbm, neigh_hbm, bn_hbm, neighs_out, self_out,
                 idx_v, neigh_v, self_v, sem_n, sem_s):
    wid = lax.axis_index("s") * NC + lax.axis_index("c")
    base = wid * BPW
    pltpu.sync_copy(bn_hbm.at[wid], idx_v)

    def chunk_body(c, carry):
        off = base + c * C
        ncp = pltpu.async_copy(neigh_hbm.at[idx_v.at[c]], neigh_v, sem_n)
        scp = pltpu.async_copy(inp_hbm.at[idx_v.at[c]], self_v, sem_s)
        ncp.wait()
        pltpu.sync_copy(neigh_v, neighs_out.at[pl.ds(off, C)])
        scp.wait()
        pltpu.sync_copy(self_v, self_out.at[pl.ds(off, C)])
        return carry

    lax.fori_loop(0, NCHUNK, chunk_body, 0)


def _make_stage1():
    mesh = plsc.VectorSubcoreMesh(**_MESH)
    return pl.kernel(
        _stage1_body,
        out_type=[
            jax.ShapeDtypeStruct((BP, KPAD), jnp.int32),
            jax.ShapeDtypeStruct((BP, D), jnp.float32),
        ],
        mesh=mesh,
        scratch_types=[
            pltpu.VMEM((NCHUNK, C), jnp.int32),
            pltpu.VMEM((C, KPAD), jnp.int32),
            pltpu.VMEM((C, D), jnp.float32),
            pltpu.SemaphoreType.DMA,
            pltpu.SemaphoreType.DMA,
        ],
        name="sage_sc_stage1",
    )


def _stage2_body(inp_hbm, neighsT_hbm, sum_out,
                 klist_v, acc_v, sem_k, sem_rows):
    wid = lax.axis_index("s") * NC + lax.axis_index("c")
    base = wid * BPW

    def chunk_body(c, carry):
        off = base + c * C
        kcps = [
            pltpu.async_copy(neighsT_hbm.at[k, pl.ds(off, C)],
                             klist_v.at[k], sem_k)
            for k in range(K)
        ]
        for cp in kcps:
            cp.wait()
        pltpu.async_copy(inp_hbm.at[klist_v.at[0]], acc_v, sem_rows).wait()
        cps = [
            pltpu.async_copy(inp_hbm.at[klist_v.at[k]], acc_v, sem_rows,
                             add=True)
            for k in range(1, K)
        ]
        for cp in cps:
            cp.wait()
        pltpu.sync_copy(acc_v, sum_out.at[pl.ds(off, C)])
        return carry

    lax.fori_loop(0, NCHUNK, chunk_body, 0)


def _make_stage2():
    mesh = plsc.VectorSubcoreMesh(**_MESH)
    return pl.kernel(
        _stage2_body,
        out_type=jax.ShapeDtypeStruct((BP, D), jnp.float32),
        mesh=mesh,
        scratch_types=[
            pltpu.VMEM((K, C), jnp.int32),
            pltpu.VMEM((C, D), jnp.float32),
            pltpu.SemaphoreType.DMA,
            pltpu.SemaphoreType.DMA,
        ],
        name="sage_sc_stage2",
    )


def _mm_body(self_ref, sum_ref, w1_ref, w2_ref, o_ref):
    a = self_ref[...]
    m = sum_ref[...] * jnp.float32(1.0 / K)
    acc = jnp.dot(a, w1_ref[...], preferred_element_type=jnp.float32)
    acc += jnp.dot(m, w2_ref[...], preferred_element_type=jnp.float32)
    o_ref[...] = jnp.maximum(acc, 0.0)


def _tc_matmul(self_rows, neigh_sum, w1, w2):
    grid = (BP // ROWS_TC,)
    return pl.pallas_call(
        _mm_body,
        grid=grid,
        in_specs=[
            pl.BlockSpec((ROWS_TC, D), lambda i: (i, 0)),
            pl.BlockSpec((ROWS_TC, D), lambda i: (i, 0)),
            pl.BlockSpec((D, D), lambda i: (0, 0)),
            pl.BlockSpec((D, D), lambda i: (0, 0)),
        ],
        out_specs=pl.BlockSpec((ROWS_TC, D), lambda i: (i, 0)),
        out_shape=jax.ShapeDtypeStruct((BP, D), jnp.float32),
    )(self_rows, neigh_sum, w1, w2)


@jax.jit
def kernel(_input, neigh_tab, batch_nodes, weight):
    neigh_tab = neigh_tab.astype(jnp.int32)
    batch_nodes = batch_nodes.astype(jnp.int32)
    neigh16 = jnp.pad(neigh_tab, ((0, 0), (0, KPAD - K)))
    bn = jnp.pad(batch_nodes, (0, BP - B)).reshape(NW, NCHUNK, C)
    neighs, self_rows = _make_stage1()(_input, neigh16, bn)
    neighsT = neighs[:, :K].T
    neigh_sum = _make_stage2()(_input, neighsT)
    out = _tc_matmul(self_rows, neigh_sum, weight[:D], weight[D:])
    return out[:B]


# speedup vs baseline: 5.8017x; 2.4988x over previous
import jax
import jax.numpy as jnp
from jax import lax
from jax.experimental import pallas as pl
from jax.experimental.pallas import tpu as pltpu
from jax.experimental.pallas import tpu_sc as plsc

N_NODES = 100000
D = 128
K = 10
B = 50000

NC = 2
NS = 16
NW = NC * NS

C = 112
NCHUNK = 14
BPW = C * NCHUNK
BP = BPW * NW

KPAD = 128

ROWS_TC = 512

_MESH = dict(core_axis_name="c", subcore_axis_name="s",
             num_cores=NC, num_subcores=NS)


def _stage1_body(inp_hbm, neigh_hbm, bn_hbm, neighs_out, self_out,
                 idx_v, neigh_v0, neigh_v1, self_v0, self_v1,
                 sem_n0, sem_n1, sem_s0, sem_s1):
    wid = lax.axis_index("s") * NC + lax.axis_index("c")
    base = wid * BPW
    pltpu.sync_copy(bn_hbm.at[wid], idx_v)

    neigh_v = (neigh_v0, neigh_v1)
    self_v = (self_v0, self_v1)
    sem_n = (sem_n0, sem_n1)
    sem_s = (sem_s0, sem_s1)

    pltpu.async_copy(neigh_hbm.at[idx_v.at[0]], neigh_v[0], sem_n[0])
    pltpu.async_copy(inp_hbm.at[idx_v.at[0]], self_v[0], sem_s[0])

    def pair_body(p, carry):
        for b in range(2):
            c = 2 * p + b
            nb = 1 - b

            @pl.when(c + 1 < NCHUNK)
            def _():
                pltpu.async_copy(neigh_hbm.at[idx_v.at[c + 1]],
                                 neigh_v[nb], sem_n[nb])
                pltpu.async_copy(inp_hbm.at[idx_v.at[c + 1]],
                                 self_v[nb], sem_s[nb])

            off = base + c * C
            pltpu.make_async_copy(neigh_hbm.at[idx_v.at[c]],
                                  neigh_v[b], sem_n[b]).wait()
            pltpu.sync_copy(neigh_v[b], neighs_out.at[pl.ds(off, C)])
            pltpu.make_async_copy(inp_hbm.at[idx_v.at[c]],
                                  self_v[b], sem_s[b]).wait()
            pltpu.sync_copy(self_v[b], self_out.at[pl.ds(off, C)])
        return carry

    lax.fori_loop(0, NCHUNK // 2, pair_body, 0)


def _make_stage1():
    mesh = plsc.VectorSubcoreMesh(**_MESH)
    return pl.kernel(
        _stage1_body,
        out_type=[
            jax.ShapeDtypeStruct((BP, KPAD), jnp.int32),
            jax.ShapeDtypeStruct((BP, D), jnp.float32),
        ],
        mesh=mesh,
        scratch_types=[
            pltpu.VMEM((NCHUNK, C), jnp.int32),
            pltpu.VMEM((C, KPAD), jnp.int32),
            pltpu.VMEM((C, KPAD), jnp.int32),
            pltpu.VMEM((C, D), jnp.float32),
            pltpu.VMEM((C, D), jnp.float32),
            pltpu.SemaphoreType.DMA,
            pltpu.SemaphoreType.DMA,
            pltpu.SemaphoreType.DMA,
            pltpu.SemaphoreType.DMA,
        ],
        name="sage_sc_stage1",
    )


def _stage2_body(inp_hbm, neighsT_hbm, sum_out,
                 klist_v, acc_v0, acc_v1,
                 sem_k, sem_r0, sem_r1):
    wid = lax.axis_index("s") * NC + lax.axis_index("c")
    base = wid * BPW

    acc_v = (acc_v0, acc_v1)
    sem_r = (sem_r0, sem_r1)

    for k in range(K):
        pltpu.async_copy(neighsT_hbm.at[k, wid], klist_v.at[k], sem_k)
    for k in range(K):
        pltpu.make_async_copy(neighsT_hbm.at[k, wid], klist_v.at[k],
                              sem_k).wait()

    zeros16 = jnp.zeros((16,), jnp.float32)

    def zero_acc(b):
        def zero_row(r, carry2):
            for l in range(D // 16):
                acc_v[b][r, pl.ds(16 * l, 16)] = zeros16
            return carry2
        lax.fori_loop(0, C, zero_row, 0)

    def fire_adds(c, b):
        for k in range(K):
            pltpu.async_copy(inp_hbm.at[klist_v.at[k, c]], acc_v[b],
                             sem_r[b], add=True)

    def drain_adds(c, b):
        for k in range(K):
            pltpu.make_async_copy(inp_hbm.at[klist_v.at[k, c]], acc_v[b],
                                  sem_r[b]).wait()

    zero_acc(0)
    fire_adds(0, 0)

    def pair_body(p, carry):
        for b in range(2):
            c = 2 * p + b
            nb = 1 - b

            @pl.when(c + 1 < NCHUNK)
            def _():
                zero_acc(nb)
                fire_adds(c + 1, nb)

            drain_adds(c, b)
            off = base + c * C
            pltpu.sync_copy(acc_v[b], sum_out.at[pl.ds(off, C)])
        return carry

    lax.fori_loop(0, NCHUNK // 2, pair_body, 0)


def _make_stage2():
    mesh = plsc.VectorSubcoreMesh(**_MESH)
    return pl.kernel(
        _stage2_body,
        out_type=jax.ShapeDtypeStruct((BP, D), jnp.float32),
        mesh=mesh,
        scratch_types=[
            pltpu.VMEM((K, NCHUNK, C), jnp.int32),
            pltpu.VMEM((C, D), jnp.float32),
            pltpu.VMEM((C, D), jnp.float32),
            pltpu.SemaphoreType.DMA,
            pltpu.SemaphoreType.DMA,
            pltpu.SemaphoreType.DMA,
        ],
        name="sage_sc_stage2",
    )


def _mm_body(self_ref, sum_ref, w1_ref, w2_ref, o_ref):
    a = self_ref[...]
    m = sum_ref[...] * jnp.float32(1.0 / K)
    acc = jnp.dot(a, w1_ref[...], preferred_element_type=jnp.float32)
    acc += jnp.dot(m, w2_ref[...], preferred_element_type=jnp.float32)
    o_ref[...] = jnp.maximum(acc, 0.0)


def _tc_matmul(self_rows, neigh_sum, w1, w2):
    grid = (BP // ROWS_TC,)
    return pl.pallas_call(
        _mm_body,
        grid=grid,
        in_specs=[
            pl.BlockSpec((ROWS_TC, D), lambda i: (i, 0)),
            pl.BlockSpec((ROWS_TC, D), lambda i: (i, 0)),
            pl.BlockSpec((D, D), lambda i: (0, 0)),
            pl.BlockSpec((D, D), lambda i: (0, 0)),
        ],
        out_specs=pl.BlockSpec((ROWS_TC, D), lambda i: (i, 0)),
        out_shape=jax.ShapeDtypeStruct((BP, D), jnp.float32),
    )(self_rows, neigh_sum, w1, w2)


@jax.jit
def kernel(_input, neigh_tab, batch_nodes, weight):
    neigh_tab = neigh_tab.astype(jnp.int32)
    batch_nodes = batch_nodes.astype(jnp.int32)
    neigh128 = jnp.pad(neigh_tab, ((0, 0), (0, KPAD - K)))
    bn = jnp.pad(batch_nodes, (0, BP - B)).reshape(NW, NCHUNK, C)
    neighs, self_rows = _make_stage1()(_input, neigh128, bn)
    neighsT = neighs[:, :K].T.reshape(K, NW, NCHUNK, C)
    neigh_sum = _make_stage2()(_input, neighsT)
    out = _tc_matmul(self_rows, neigh_sum, weight[:D], weight[D:])
    return out[:B]


# speedup vs baseline: 5.8547x; 1.0091x over previous
import jax
import jax.numpy as jnp
from jax import lax
from jax.experimental import pallas as pl
from jax.experimental.pallas import tpu as pltpu
from jax.experimental.pallas import tpu_sc as plsc

N_NODES = 100000
D = 128
K = 10
B = 50000

NC = 2
NS = 16
NW = NC * NS

C = 112
NCHUNK = 14
BPW = C * NCHUNK
BP = BPW * NW

KPAD = 128

ROWS_TC = 400

_MESH = dict(core_axis_name="c", subcore_axis_name="s",
             num_cores=NC, num_subcores=NS)


def _stage1_body(inp_hbm, neigh_hbm, bn_hbm, neighs_out, self_out,
                 idx_v, neigh_v0, neigh_v1, self_v0, self_v1,
                 sem_n0, sem_n1, sem_s0, sem_s1):
    wid = lax.axis_index("s") * NC + lax.axis_index("c")
    base = wid * BPW
    pltpu.sync_copy(bn_hbm.at[wid], idx_v)

    neigh_v = (neigh_v0, neigh_v1)
    self_v = (self_v0, self_v1)
    sem_n = (sem_n0, sem_n1)
    sem_s = (sem_s0, sem_s1)

    pltpu.async_copy(neigh_hbm.at[idx_v.at[0]], neigh_v[0], sem_n[0])
    pltpu.async_copy(inp_hbm.at[idx_v.at[0]], self_v[0], sem_s[0])

    def pair_body(p, carry):
        for b in range(2):
            c = 2 * p + b
            nb = 1 - b

            @pl.when(c + 1 < NCHUNK)
            def _():
                pltpu.async_copy(neigh_hbm.at[idx_v.at[c + 1]],
                                 neigh_v[nb], sem_n[nb])
                pltpu.async_copy(inp_hbm.at[idx_v.at[c + 1]],
                                 self_v[nb], sem_s[nb])

            off = base + c * C
            pltpu.make_async_copy(neigh_hbm.at[idx_v.at[c]],
                                  neigh_v[b], sem_n[b]).wait()
            pltpu.sync_copy(neigh_v[b], neighs_out.at[pl.ds(off, C)])
            pltpu.make_async_copy(inp_hbm.at[idx_v.at[c]],
                                  self_v[b], sem_s[b]).wait()
            pltpu.sync_copy(self_v[b], self_out.at[pl.ds(off, C)])
        return carry

    lax.fori_loop(0, NCHUNK // 2, pair_body, 0)


def _make_stage1():
    mesh = plsc.VectorSubcoreMesh(**_MESH)
    return pl.kernel(
        _stage1_body,
        out_type=[
            jax.ShapeDtypeStruct((BP, KPAD), jnp.int32),
            jax.ShapeDtypeStruct((BP, D), jnp.float32),
        ],
        mesh=mesh,
        scratch_types=[
            pltpu.VMEM((NCHUNK, C), jnp.int32),
            pltpu.VMEM((C, KPAD), jnp.int32),
            pltpu.VMEM((C, KPAD), jnp.int32),
            pltpu.VMEM((C, D), jnp.float32),
            pltpu.VMEM((C, D), jnp.float32),
            pltpu.SemaphoreType.DMA,
            pltpu.SemaphoreType.DMA,
            pltpu.SemaphoreType.DMA,
            pltpu.SemaphoreType.DMA,
        ],
        name="sage_sc_stage1",
    )


def _stage2_body(inp_hbm, neighsT_hbm, sum_out,
                 klist_v, acc_v0, acc_v1,
                 sem_k, sem_r0, sem_r1):
    wid = lax.axis_index("s") * NC + lax.axis_index("c")
    base = wid * BPW

    acc_v = (acc_v0, acc_v1)
    sem_r = (sem_r0, sem_r1)

    for k in range(K):
        pltpu.async_copy(neighsT_hbm.at[k, wid], klist_v.at[k], sem_k)
    for k in range(K):
        pltpu.make_async_copy(neighsT_hbm.at[k, wid], klist_v.at[k],
                              sem_k).wait()

    zeros16 = jnp.zeros((16,), jnp.float32)

    def zero_acc(b):
        def zero_row(r, carry2):
            for l in range(D // 16):
                acc_v[b][r, pl.ds(16 * l, 16)] = zeros16
            return carry2
        lax.fori_loop(0, C, zero_row, 0)

    def fire_adds(c, b):
        for k in range(K):
            pltpu.async_copy(inp_hbm.at[klist_v.at[k, c]], acc_v[b],
                             sem_r[b], add=True)

    def drain_adds(c, b):
        for k in range(K):
            pltpu.make_async_copy(inp_hbm.at[klist_v.at[k, c]], acc_v[b],
                                  sem_r[b]).wait()

    zero_acc(0)
    fire_adds(0, 0)

    def pair_body(p, carry):
        for b in range(2):
            c = 2 * p + b
            nb = 1 - b

            @pl.when(c + 1 < NCHUNK)
            def _():
                zero_acc(nb)
                fire_adds(c + 1, nb)

            drain_adds(c, b)
            off = base + c * C
            pltpu.sync_copy(acc_v[b], sum_out.at[pl.ds(off, C)])
        return carry

    lax.fori_loop(0, NCHUNK // 2, pair_body, 0)


def _make_stage2():
    mesh = plsc.VectorSubcoreMesh(**_MESH)
    return pl.kernel(
        _stage2_body,
        out_type=jax.ShapeDtypeStruct((BP, D), jnp.float32),
        mesh=mesh,
        scratch_types=[
            pltpu.VMEM((K, NCHUNK, C), jnp.int32),
            pltpu.VMEM((C, D), jnp.float32),
            pltpu.VMEM((C, D), jnp.float32),
            pltpu.SemaphoreType.DMA,
            pltpu.SemaphoreType.DMA,
            pltpu.SemaphoreType.DMA,
        ],
        name="sage_sc_stage2",
    )


def _mm_body(self_ref, sum_ref, w1_ref, w2_ref, o_ref):
    a = self_ref[...]
    m = sum_ref[...] * jnp.float32(1.0 / K)
    acc = jnp.dot(a, w1_ref[...], preferred_element_type=jnp.float32)
    acc += jnp.dot(m, w2_ref[...], preferred_element_type=jnp.float32)
    o_ref[...] = jnp.maximum(acc, 0.0)


def _tc_matmul(self_rows, neigh_sum, w1, w2):
    grid = (B // ROWS_TC,)
    return pl.pallas_call(
        _mm_body,
        grid=grid,
        in_specs=[
            pl.BlockSpec((ROWS_TC, D), lambda i: (i, 0)),
            pl.BlockSpec((ROWS_TC, D), lambda i: (i, 0)),
            pl.BlockSpec((D, D), lambda i: (0, 0)),
            pl.BlockSpec((D, D), lambda i: (0, 0)),
        ],
        out_specs=pl.BlockSpec((ROWS_TC, D), lambda i: (i, 0)),
        out_shape=jax.ShapeDtypeStruct((B, D), jnp.float32),
    )(self_rows, neigh_sum, w1, w2)


@jax.jit
def kernel(_input, neigh_tab, batch_nodes, weight):
    neigh_tab = neigh_tab.astype(jnp.int32)
    batch_nodes = batch_nodes.astype(jnp.int32)
    neigh128 = jnp.pad(neigh_tab, ((0, 0), (0, KPAD - K)))
    bn = jnp.pad(batch_nodes, (0, BP - B)).reshape(NW, NCHUNK, C)
    neighs, self_rows = _make_stage1()(_input, neigh128, bn)
    neighsT = neighs[:, :K].T.reshape(K, NW, NCHUNK, C)
    neigh_sum = _make_stage2()(_input, neighsT)
    return _tc_matmul(self_rows, neigh_sum, weight[:D], weight[D:])


# speedup vs baseline: 6.5296x; 1.1153x over previous
import jax
import jax.numpy as jnp
from jax import lax
from jax.experimental import pallas as pl
from jax.experimental.pallas import tpu as pltpu
from jax.experimental.pallas import tpu_sc as plsc

N_NODES = 100000
D = 128
K = 10
B = 50000

NC = 2
NS = 16
NW = NC * NS

C = 112
NCHUNK = 14
BPW = C * NCHUNK
BP = BPW * NW

KPAD = 128

ROWS_TC = 1000

_MESH = dict(core_axis_name="c", subcore_axis_name="s",
             num_cores=NC, num_subcores=NS)


def _stage1_body(inp_hbm, neigh_hbm, bn_hbm, neighs_out, self_out,
                 idx_v, neigh_v0, neigh_v1, self_v0, self_v1,
                 sem_n0, sem_n1, sem_s0, sem_s1):
    wid = lax.axis_index("s") * NC + lax.axis_index("c")
    base = wid * BPW
    pltpu.sync_copy(bn_hbm.at[wid], idx_v)

    neigh_v = (neigh_v0, neigh_v1)
    self_v = (self_v0, self_v1)
    sem_n = (sem_n0, sem_n1)
    sem_s = (sem_s0, sem_s1)

    pltpu.async_copy(neigh_hbm.at[idx_v.at[0]], neigh_v[0], sem_n[0])
    pltpu.async_copy(inp_hbm.at[idx_v.at[0]], self_v[0], sem_s[0])

    def pair_body(p, carry):
        for b in range(2):
            c = 2 * p + b
            nb = 1 - b

            @pl.when(c + 1 < NCHUNK)
            def _():
                pltpu.async_copy(neigh_hbm.at[idx_v.at[c + 1]],
                                 neigh_v[nb], sem_n[nb])
                pltpu.async_copy(inp_hbm.at[idx_v.at[c + 1]],
                                 self_v[nb], sem_s[nb])

            off = base + c * C
            pltpu.make_async_copy(neigh_hbm.at[idx_v.at[c]],
                                  neigh_v[b], sem_n[b]).wait()
            pltpu.sync_copy(neigh_v[b], neighs_out.at[pl.ds(off, C)])
            pltpu.make_async_copy(inp_hbm.at[idx_v.at[c]],
                                  self_v[b], sem_s[b]).wait()
            pltpu.sync_copy(self_v[b], self_out.at[pl.ds(off, C)])
        return carry

    lax.fori_loop(0, NCHUNK // 2, pair_body, 0)


def _make_stage1():
    mesh = plsc.VectorSubcoreMesh(**_MESH)
    return pl.kernel(
        _stage1_body,
        out_type=[
            jax.ShapeDtypeStruct((BP, KPAD), jnp.int32),
            jax.ShapeDtypeStruct((BP, D), jnp.float32),
        ],
        mesh=mesh,
        scratch_types=[
            pltpu.VMEM((NCHUNK, C), jnp.int32),
            pltpu.VMEM((C, KPAD), jnp.int32),
            pltpu.VMEM((C, KPAD), jnp.int32),
            pltpu.VMEM((C, D), jnp.float32),
            pltpu.VMEM((C, D), jnp.float32),
            pltpu.SemaphoreType.DMA,
            pltpu.SemaphoreType.DMA,
            pltpu.SemaphoreType.DMA,
            pltpu.SemaphoreType.DMA,
        ],
        name="sage_sc_stage1",
    )


def _stage2_body(inp_hbm, neighsT_hbm, sum_out,
                 klist_v, acc_v0, acc_v1,
                 sem_k, sem_r0, sem_r1):
    wid = lax.axis_index("s") * NC + lax.axis_index("c")
    base = wid * BPW

    acc_v = (acc_v0, acc_v1)
    sem_r = (sem_r0, sem_r1)

    for k in range(K):
        pltpu.async_copy(neighsT_hbm.at[k, wid], klist_v.at[k], sem_k)
    for k in range(K):
        pltpu.make_async_copy(neighsT_hbm.at[k, wid], klist_v.at[k],
                              sem_k).wait()

    zeros16 = jnp.zeros((16,), jnp.float32)

    def zero_acc(b):
        def zero_row(r, carry2):
            for l in range(D // 16):
                acc_v[b][r, pl.ds(16 * l, 16)] = zeros16
            return carry2
        lax.fori_loop(0, C, zero_row, 0)

    def fire_adds(c, b):
        for k in range(K):
            pltpu.async_copy(inp_hbm.at[klist_v.at[k, c]], acc_v[b],
                             sem_r[b], add=True)

    def drain_adds(c, b):
        for k in range(K):
            pltpu.make_async_copy(inp_hbm.at[klist_v.at[k, c]], acc_v[b],
                                  sem_r[b]).wait()

    zero_acc(0)
    fire_adds(0, 0)

    def pair_body(p, carry):
        for b in range(2):
            c = 2 * p + b
            nb = 1 - b

            @pl.when(c + 1 < NCHUNK)
            def _():
                zero_acc(nb)
                fire_adds(c + 1, nb)

            drain_adds(c, b)
            off = base + c * C
            pltpu.sync_copy(acc_v[b], sum_out.at[pl.ds(off, C)])
        return carry

    lax.fori_loop(0, NCHUNK // 2, pair_body, 0)


def _make_stage2():
    mesh = plsc.VectorSubcoreMesh(**_MESH)
    return pl.kernel(
        _stage2_body,
        out_type=jax.ShapeDtypeStruct((BP, D), jnp.float32),
        mesh=mesh,
        scratch_types=[
            pltpu.VMEM((K, NCHUNK, C), jnp.int32),
            pltpu.VMEM((C, D), jnp.float32),
            pltpu.VMEM((C, D), jnp.float32),
            pltpu.SemaphoreType.DMA,
            pltpu.SemaphoreType.DMA,
            pltpu.SemaphoreType.DMA,
        ],
        name="sage_sc_stage2",
    )


def _mm_body(self_ref, sum_ref, w1_ref, w2_ref, o_ref):
    a = self_ref[...]
    m = sum_ref[...] * jnp.float32(1.0 / K)
    acc = jnp.dot(a, w1_ref[...], preferred_element_type=jnp.float32,
                  precision=lax.Precision.DEFAULT)
    acc += jnp.dot(m, w2_ref[...], preferred_element_type=jnp.float32,
                   precision=lax.Precision.DEFAULT)
    o_ref[...] = jnp.maximum(acc, 0.0)


def _tc_matmul(self_rows, neigh_sum, w1, w2):
    grid = (B // ROWS_TC,)
    return pl.pallas_call(
        _mm_body,
        grid=grid,
        in_specs=[
            pl.BlockSpec((ROWS_TC, D), lambda i: (i, 0)),
            pl.BlockSpec((ROWS_TC, D), lambda i: (i, 0)),
            pl.BlockSpec((D, D), lambda i: (0, 0)),
            pl.BlockSpec((D, D), lambda i: (0, 0)),
        ],
        out_specs=pl.BlockSpec((ROWS_TC, D), lambda i: (i, 0)),
        out_shape=jax.ShapeDtypeStruct((B, D), jnp.float32),
    )(self_rows, neigh_sum, w1, w2)


@jax.jit
def kernel(_input, neigh_tab, batch_nodes, weight):
    neigh_tab = neigh_tab.astype(jnp.int32)
    batch_nodes = batch_nodes.astype(jnp.int32)
    neigh128 = jnp.pad(neigh_tab, ((0, 0), (0, KPAD - K)))
    bn = jnp.pad(batch_nodes, (0, BP - B)).reshape(NW, NCHUNK, C)
    neighs, self_rows = _make_stage1()(_input, neigh128, bn)
    neighsT = neighs[:, :K].T.reshape(K, NW, NCHUNK, C)
    neigh_sum = _make_stage2()(_input, neighsT)
    return _tc_matmul(self_rows, neigh_sum, weight[:D], weight[D:])


# speedup vs baseline: 6.5366x; 1.0011x over previous
import jax
import jax.numpy as jnp
from jax import lax
from jax.experimental import pallas as pl
from jax.experimental.pallas import tpu as pltpu
from jax.experimental.pallas import tpu_sc as plsc

N_NODES = 100000
D = 128
K = 10
B = 50000

NC = 2
NS = 16
NW = NC * NS

C = 112
NCHUNK = 14
BPW = C * NCHUNK
BP = BPW * NW

KPAD = 128

ROWS_TC = 1000

_MESH = dict(core_axis_name="c", subcore_axis_name="s",
             num_cores=NC, num_subcores=NS)


def _stage1_body(inp_hbm, neigh_hbm, bn_hbm, neighs_out, self_out,
                 idx_v, neigh_v0, neigh_v1, self_v0, self_v1,
                 sem_n0, sem_n1, sem_s0, sem_s1):
    wid = lax.axis_index("s") * NC + lax.axis_index("c")
    base = wid * BPW
    pltpu.sync_copy(bn_hbm.at[wid], idx_v)

    neigh_v = (neigh_v0, neigh_v1)
    self_v = (self_v0, self_v1)
    sem_n = (sem_n0, sem_n1)
    sem_s = (sem_s0, sem_s1)

    pltpu.async_copy(neigh_hbm.at[idx_v.at[0]], neigh_v[0], sem_n[0])
    pltpu.async_copy(inp_hbm.at[idx_v.at[0]], self_v[0], sem_s[0])

    def pair_body(p, carry):
        for b in range(2):
            c = 2 * p + b
            nb = 1 - b

            @pl.when(c + 1 < NCHUNK)
            def _():
                pltpu.async_copy(neigh_hbm.at[idx_v.at[c + 1]],
                                 neigh_v[nb], sem_n[nb])
                pltpu.async_copy(inp_hbm.at[idx_v.at[c + 1]],
                                 self_v[nb], sem_s[nb])

            off = base + c * C
            pltpu.make_async_copy(neigh_hbm.at[idx_v.at[c]],
                                  neigh_v[b], sem_n[b]).wait()
            pltpu.sync_copy(neigh_v[b], neighs_out.at[pl.ds(off, C)])
            pltpu.make_async_copy(inp_hbm.at[idx_v.at[c]],
                                  self_v[b], sem_s[b]).wait()
            pltpu.sync_copy(self_v[b], self_out.at[pl.ds(off, C)])
        return carry

    lax.fori_loop(0, NCHUNK // 2, pair_body, 0)


def _make_stage1():
    mesh = plsc.VectorSubcoreMesh(**_MESH)
    return pl.kernel(
        _stage1_body,
        out_type=[
            jax.ShapeDtypeStruct((BP, KPAD), jnp.int32),
            jax.ShapeDtypeStruct((BP, D), jnp.float32),
        ],
        mesh=mesh,
        scratch_types=[
            pltpu.VMEM((NCHUNK, C), jnp.int32),
            pltpu.VMEM((C, KPAD), jnp.int32),
            pltpu.VMEM((C, KPAD), jnp.int32),
            pltpu.VMEM((C, D), jnp.float32),
            pltpu.VMEM((C, D), jnp.float32),
            pltpu.SemaphoreType.DMA,
            pltpu.SemaphoreType.DMA,
            pltpu.SemaphoreType.DMA,
            pltpu.SemaphoreType.DMA,
        ],
        name="sage_sc_stage1",
    )


def _stage2_body(inp_hbm, neighsT_hbm, sum_out,
                 klist_v, acc_v0, acc_v1,
                 sem_k, sem_r0, sem_r1):
    wid = lax.axis_index("s") * NC + lax.axis_index("c")
    base = wid * BPW

    acc_v = (acc_v0, acc_v1)
    sem_r = (sem_r0, sem_r1)

    for k in range(K):
        pltpu.async_copy(neighsT_hbm.at[k, wid], klist_v.at[k], sem_k)
    for k in range(K):
        pltpu.make_async_copy(neighsT_hbm.at[k, wid], klist_v.at[k],
                              sem_k).wait()

    zeros16 = jnp.zeros((16,), jnp.float32)

    def zero_acc(b):
        def zero_row(r, carry2):
            for l in range(D // 16):
                acc_v[b][r, pl.ds(16 * l, 16)] = zeros16
            return carry2
        lax.fori_loop(0, C, zero_row, 0)

    def fire_adds(c, b):
        for k in range(K):
            pltpu.async_copy(inp_hbm.at[klist_v.at[k, c]], acc_v[b],
                             sem_r[b], add=True)

    def drain_adds(c, b):
        for k in range(K):
            pltpu.make_async_copy(inp_hbm.at[klist_v.at[k, c]], acc_v[b],
                                  sem_r[b]).wait()

    zero_acc(0)
    fire_adds(0, 0)

    def pair_body(p, carry):
        for b in range(2):
            c = 2 * p + b
            nb = 1 - b

            @pl.when(c + 1 < NCHUNK)
            def _():
                zero_acc(nb)
                fire_adds(c + 1, nb)

            drain_adds(c, b)
            off = base + c * C
            pltpu.sync_copy(acc_v[b], sum_out.at[pl.ds(off, C)])
        return carry

    lax.fori_loop(0, NCHUNK // 2, pair_body, 0)


def _make_stage2():
    mesh = plsc.VectorSubcoreMesh(**_MESH)
    return pl.kernel(
        _stage2_body,
        out_type=jax.ShapeDtypeStruct((BP, D), jnp.float32),
        mesh=mesh,
        scratch_types=[
            pltpu.VMEM((K, NCHUNK, C), jnp.int32),
            pltpu.VMEM((C, D), jnp.float32),
            pltpu.VMEM((C, D), jnp.float32),
            pltpu.SemaphoreType.DMA,
            pltpu.SemaphoreType.DMA,
            pltpu.SemaphoreType.DMA,
        ],
        name="sage_sc_stage2",
    )


def _mm_body(self_ref, sum_ref, w1_ref, w2_ref, o_ref):
    a = self_ref[...].astype(jnp.bfloat16)
    m = (sum_ref[...] * jnp.float32(1.0 / K)).astype(jnp.bfloat16)
    w1 = w1_ref[...].astype(jnp.bfloat16)
    w2 = w2_ref[...].astype(jnp.bfloat16)
    acc = jnp.dot(a, w1, preferred_element_type=jnp.float32)
    acc += jnp.dot(m, w2, preferred_element_type=jnp.float32)
    o_ref[...] = jnp.maximum(acc, 0.0)


def _tc_matmul(self_rows, neigh_sum, w1, w2):
    grid = (B // ROWS_TC,)
    return pl.pallas_call(
        _mm_body,
        grid=grid,
        in_specs=[
            pl.BlockSpec((ROWS_TC, D), lambda i: (i, 0)),
            pl.BlockSpec((ROWS_TC, D), lambda i: (i, 0)),
            pl.BlockSpec((D, D), lambda i: (0, 0)),
            pl.BlockSpec((D, D), lambda i: (0, 0)),
        ],
        out_specs=pl.BlockSpec((ROWS_TC, D), lambda i: (i, 0)),
        out_shape=jax.ShapeDtypeStruct((B, D), jnp.float32),
    )(self_rows, neigh_sum, w1, w2)


@jax.jit
def kernel(_input, neigh_tab, batch_nodes, weight):
    neigh_tab = neigh_tab.astype(jnp.int32)
    batch_nodes = batch_nodes.astype(jnp.int32)
    neigh128 = jnp.pad(neigh_tab, ((0, 0), (0, KPAD - K)))
    bn = jnp.pad(batch_nodes, (0, BP - B)).reshape(NW, NCHUNK, C)
    neighs, self_rows = _make_stage1()(_input, neigh128, bn)
    neighsT = neighs[:, :K].T.reshape(K, NW, NCHUNK, C)
    neigh_sum = _make_stage2()(_input, neighsT)
    return _tc_matmul(self_rows, neigh_sum, weight[:D], weight[D:])
